# Initial kernel scaffold; baseline (speedup 1.0000x reference)
import jax
import jax.numpy as jnp
from jax.experimental import pallas as pl


def kernel(x, W_emb, b_emb, W_lins, b_conv, W_ro, b_ro):
    raise NotImplementedError("write your pallas kernel here")


# speedup vs baseline: 7.5139x; 7.5139x over previous
import functools

import jax
import jax.numpy as jnp
from jax import lax
from jax.experimental import pallas as pl
from jax.experimental.pallas import tpu as pltpu
from jax.experimental.pallas import tpu_sc as plsc

_N = 10000
_NPAD = 10240
_DH = 128
_KNN = 10
_STEPS = 6
_EPS = 0.1

_ROWB = 128
_NBLK = _NPAD // _ROWB

_NW = 32
_BPW = _NPAD // _NW
_CHUNK = 64
_NCH = _BPW // _CHUNK
_EPC = _CHUNK * _KNN
_GB = 128
_NGATH = _EPC // _GB

def _knn_body(xr_ref, xt_ref, sqr_ref, sqc_ref, out_ref):
    i = pl.program_id(0)
    xr = xr_ref[...]
    xt = xt_ref[...]
    dot = jnp.dot(xr, xt, preferred_element_type=jnp.float32)
    d2 = (sqr_ref[...] + sqc_ref[...]) - 2.0 * dot
    col = lax.broadcasted_iota(jnp.int32, (_ROWB, _NPAD), 1)
    row = lax.broadcasted_iota(jnp.int32, (_ROWB, _NPAD), 0) + i * _ROWB
    inf = jnp.float32(jnp.inf)
    d2 = jnp.where((col == row) | (col >= _N), inf, d2)
    big = jnp.int32(2 ** 30)
    out_col = lax.broadcasted_iota(jnp.int32, (_ROWB, 16), 1)
    out = jnp.zeros((_ROWB, 16), jnp.int32)
    for t in range(_KNN):
        m = jnp.min(d2, axis=1, keepdims=True)
        idx = jnp.min(jnp.where(d2 == m, col, big), axis=1, keepdims=True)
        out = jnp.where(out_col == t, idx, out)
        if t < _KNN - 1:
            d2 = jnp.where(col == idx, inf, d2)
    out_ref[...] = out


def _knn(x_rows, x_t, sq_col, sq_row):
    return pl.pallas_call(
        _knn_body,
        grid=(_NBLK,),
        in_specs=[
            pl.BlockSpec((_ROWB, 8), lambda i: (i, 0)),
            pl.BlockSpec((8, _NPAD), lambda i: (0, 0)),
            pl.BlockSpec((_ROWB, 1), lambda i: (i, 0)),
            pl.BlockSpec((1, _NPAD), lambda i: (0, 0)),
        ],
        out_specs=pl.BlockSpec((_ROWB, 16), lambda i: (i, 0)),
        out_shape=jax.ShapeDtypeStruct((_NPAD, 16), jnp.int32),
    )(x_rows, x_t, sq_col, sq_row)


def _init_body(xr_ref, we_ref, be_ref, wr_ref, br_ref, h_ref, x0_ref):
    h = jnp.dot(xr_ref[...], we_ref[...],
                preferred_element_type=jnp.float32) + be_ref[...]
    h_ref[...] = h
    x0_ref[...] = jnp.dot(h, wr_ref[...],
                          preferred_element_type=jnp.float32) + br_ref[...]


def _init(x_rows, w_emb8, b_emb, w_ro, b_ro):
    blk = 1280
    full = lambda i: (0, 0)
    return pl.pallas_call(
        _init_body,
        grid=(_NPAD // blk,),
        in_specs=[
            pl.BlockSpec((blk, 8), lambda i: (i, 0)),
            pl.BlockSpec((8, _DH), full),
            pl.BlockSpec((1, _DH), full),
            pl.BlockSpec((_DH, _DH), full),
            pl.BlockSpec((1, _DH), full),
        ],
        out_specs=[
            pl.BlockSpec((blk, _DH), lambda i: (i, 0)),
            pl.BlockSpec((blk, _DH), lambda i: (i, 0)),
        ],
        out_shape=[
            jax.ShapeDtypeStruct((_NPAD, _DH), jnp.float32),
            jax.ShapeDtypeStruct((_NPAD, _DH), jnp.float32),
        ],
    )(x_rows, w_emb8, b_emb, w_ro, b_ro)


def _step_body(h_ref, g1_ref, g2_ref, w0_ref, w1_ref, w2_ref, bc_ref,
               wr_ref, br_ref, nrm_ref, hn_ref, pr_ref):
    nrm = nrm_ref[0, 0]
    x1 = g1_ref[...] * nrm
    x2 = (g2_ref[...] * nrm) * nrm
    conv = jnp.dot(h_ref[...], w0_ref[...],
                   preferred_element_type=jnp.float32)
    conv = conv + jnp.dot(x1, w1_ref[...],
                          preferred_element_type=jnp.float32)
    conv = conv + jnp.dot(x2, w2_ref[...],
                          preferred_element_type=jnp.float32)
    conv = conv + bc_ref[...]
    hn = h_ref[...] + _EPS * jnp.tanh(conv)
    hn_ref[...] = hn
    pr_ref[...] = jnp.dot(hn, wr_ref[...],
                          preferred_element_type=jnp.float32) + br_ref[...]


def _step(h, g1, g2, w0, w1, w2, b_conv, w_ro, b_ro, nrm):
    blk = 1280
    full = lambda i: (0, 0)
    rows = lambda i: (i, 0)
    return pl.pallas_call(
        _step_body,
        grid=(_NPAD // blk,),
        in_specs=[
            pl.BlockSpec((blk, _DH), rows),
            pl.BlockSpec((blk, _DH), rows),
            pl.BlockSpec((blk, _DH), rows),
            pl.BlockSpec((_DH, _DH), full),
            pl.BlockSpec((_DH, _DH), full),
            pl.BlockSpec((_DH, _DH), full),
            pl.BlockSpec((1, _DH), full),
            pl.BlockSpec((_DH, _DH), full),
            pl.BlockSpec((1, _DH), full),
            pl.BlockSpec((1, 1), full),
        ],
        out_specs=[
            pl.BlockSpec((blk, _DH), rows),
            pl.BlockSpec((blk, _DH), rows),
        ],
        out_shape=[
            jax.ShapeDtypeStruct((_NPAD, _DH), jnp.float32),
            jax.ShapeDtypeStruct((_NPAD, _DH), jnp.float32),
        ],
    )(h, g1, g2, w0, w1, w2, b_conv, w_ro, b_ro, nrm)


@functools.cache
def _make_gather_sum():
    return functools.partial(
        pl.kernel,
        out_type=jax.ShapeDtypeStruct((_NPAD, _DH), jnp.float32),
        mesh=plsc.VectorSubcoreMesh(core_axis_name="c", subcore_axis_name="s"),
        scratch_types=[
            pltpu.VMEM((_BPW * _KNN,), jnp.int32),
            pltpu.VMEM((_EPC, _DH), jnp.float32),
            pltpu.VMEM((_CHUNK, _DH), jnp.float32),
            pltpu.SemaphoreType.DMA,
        ],
    )(_gather_sum_body)


def _gather_sum_body(table_hbm, idx_hbm, out_hbm, idx_v, rows_v, out_v, sem):
    wid = lax.axis_index("s") * 2 + lax.axis_index("c")
    base = wid * _BPW
    pltpu.sync_copy(idx_hbm.at[pl.ds(base * _KNN, _BPW * _KNN)], idx_v)

    def chunk_body(c, carry):
        copies = [
            pltpu.async_copy(
                table_hbm.at[idx_v.at[pl.ds(c * _EPC + j * _GB, _GB)]],
                rows_v.at[pl.ds(j * _GB, _GB)],
                sem,
            )
            for j in range(_NGATH)
        ]
        for cp in copies:
            cp.wait()

        def node_body(n, carry2):
            e0 = n * _KNN
            for j in range(_DH // 16):
                sl = pl.ds(j * 16, 16)
                acc = rows_v[e0, sl]
                for e in range(1, _KNN):
                    acc = acc + rows_v[e0 + e, sl]
                out_v[n, sl] = acc
            return carry2

        lax.fori_loop(0, _CHUNK, node_body, 0)
        pltpu.sync_copy(out_v, out_hbm.at[pl.ds(base + c * _CHUNK, _CHUNK)])
        return carry

    lax.fori_loop(0, _NCH, chunk_body, 0)


def kernel(x, W_emb, b_emb, W_lins, b_conv, W_ro, b_ro):
    f32 = jnp.float32
    x_rows = jnp.zeros((_NPAD, 8), f32).at[:_N, :3].set(x)
    x_t = x_rows.T
    sq = jnp.sum(x * x, axis=1)
    sq_pad = jnp.zeros((_NPAD,), f32).at[:_N].set(sq)

    nbr = _knn(x_rows, x_t, sq_pad.reshape(_NPAD, 1), sq_pad.reshape(1, _NPAD))
    idx_flat = nbr[:, :_KNN].reshape(-1)

    dis = 1.0 / jnp.sqrt(f32(10.0))
    nrm = (dis * dis).reshape(1, 1)
    w_emb8 = jnp.zeros((8, _DH), f32).at[:3].set(W_emb)
    w0 = W_lins[0]
    w1 = W_lins[1]
    w2 = W_lins[2]
    w_ro_p = jnp.zeros((_DH, _DH), f32).at[:, :3].set(W_ro)
    b_ro_p = jnp.zeros((1, _DH), f32).at[0, :3].set(b_ro)
    b_conv_r = b_conv.reshape(1, _DH)
    b_emb_r = b_emb.reshape(1, _DH)

    h, x0 = _init(x_rows, w_emb8, b_emb_r, w_ro_p, b_ro_p)

    preds = []
    gather_sum = _make_gather_sum()
    for _t in range(_STEPS):
        g1 = gather_sum(h, idx_flat)
        g2 = gather_sum(g1, idx_flat)
        h, pr = _step(h, g1, g2, w0, w1, w2, b_conv_r, w_ro_p, b_ro_p, nrm)
        preds.append(pr[:_N, :3])

    y = preds[-1]
    return (y, h[:_N], x0[:_N, :3], jnp.stack(preds))


# speedup vs baseline: 8.7046x; 1.1585x over previous
import functools

import jax
import jax.numpy as jnp
from jax import lax
from jax.experimental import pallas as pl
from jax.experimental.pallas import tpu as pltpu
from jax.experimental.pallas import tpu_sc as plsc

_N = 10000
_NPAD = 10240
_DH = 128
_KNN = 10
_STEPS = 6
_EPS = 0.1

_ROWB = 128
_NBLK = _NPAD // _ROWB

_NW = 32
_BPW = _NPAD // _NW
_CHUNK = 32
_NCH = _BPW // _CHUNK
_EPC = _CHUNK * _KNN
_GB = 80
_NGATH = _EPC // _GB

def _knn_body(xr_ref, xt_ref, sqr_ref, sqc_ref, out_ref):
    i = pl.program_id(0)
    xr = xr_ref[...]
    xt = xt_ref[...]
    dot = jnp.dot(xr, xt, preferred_element_type=jnp.float32)
    d2 = (sqr_ref[...] + sqc_ref[...]) - 2.0 * dot
    col = lax.broadcasted_iota(jnp.int32, (_ROWB, _NPAD), 1)
    row = lax.broadcasted_iota(jnp.int32, (_ROWB, _NPAD), 0) + i * _ROWB
    inf = jnp.float32(jnp.inf)
    d2 = jnp.where((col == row) | (col >= _N), inf, d2)
    big = jnp.int32(2 ** 30)
    out_col = lax.broadcasted_iota(jnp.int32, (_ROWB, 16), 1)
    out = jnp.zeros((_ROWB, 16), jnp.int32)
    for t in range(_KNN):
        m = jnp.min(d2, axis=1, keepdims=True)
        idx = jnp.min(jnp.where(d2 == m, col, big), axis=1, keepdims=True)
        out = jnp.where(out_col == t, idx, out)
        if t < _KNN - 1:
            d2 = jnp.where(col == idx, inf, d2)
    out_ref[...] = out


def _knn(x_rows, x_t, sq_col, sq_row):
    return pl.pallas_call(
        _knn_body,
        grid=(_NBLK,),
        in_specs=[
            pl.BlockSpec((_ROWB, 8), lambda i: (i, 0)),
            pl.BlockSpec((8, _NPAD), lambda i: (0, 0)),
            pl.BlockSpec((_ROWB, 1), lambda i: (i, 0)),
            pl.BlockSpec((1, _NPAD), lambda i: (0, 0)),
        ],
        out_specs=pl.BlockSpec((_ROWB, 16), lambda i: (i, 0)),
        out_shape=jax.ShapeDtypeStruct((_NPAD, 16), jnp.int32),
    )(x_rows, x_t, sq_col, sq_row)


def _init_body(xr_ref, we_ref, be_ref, wr_ref, br_ref, h_ref, x0_ref):
    h = jnp.dot(xr_ref[...], we_ref[...],
                preferred_element_type=jnp.float32) + be_ref[...]
    h_ref[...] = h
    x0_ref[...] = jnp.dot(h, wr_ref[...],
                          preferred_element_type=jnp.float32) + br_ref[...]


def _init(x_rows, w_emb8, b_emb, w_ro, b_ro):
    blk = 1280
    full = lambda i: (0, 0)
    return pl.pallas_call(
        _init_body,
        grid=(_NPAD // blk,),
        in_specs=[
            pl.BlockSpec((blk, 8), lambda i: (i, 0)),
            pl.BlockSpec((8, _DH), full),
            pl.BlockSpec((1, _DH), full),
            pl.BlockSpec((_DH, _DH), full),
            pl.BlockSpec((1, _DH), full),
        ],
        out_specs=[
            pl.BlockSpec((blk, _DH), lambda i: (i, 0)),
            pl.BlockSpec((blk, _DH), lambda i: (i, 0)),
        ],
        out_shape=[
            jax.ShapeDtypeStruct((_NPAD, _DH), jnp.float32),
            jax.ShapeDtypeStruct((_NPAD, _DH), jnp.float32),
        ],
    )(x_rows, w_emb8, b_emb, w_ro, b_ro)


def _step_body(h_ref, g1_ref, g2_ref, w0_ref, w1_ref, w2_ref, bc_ref,
               wr_ref, br_ref, nrm_ref, hn_ref, pr_ref):
    nrm = nrm_ref[0, 0]
    x1 = g1_ref[...] * nrm
    x2 = (g2_ref[...] * nrm) * nrm
    conv = jnp.dot(h_ref[...], w0_ref[...],
                   preferred_element_type=jnp.float32)
    conv = conv + jnp.dot(x1, w1_ref[...],
                          preferred_element_type=jnp.float32)
    conv = conv + jnp.dot(x2, w2_ref[...],
                          preferred_element_type=jnp.float32)
    conv = conv + bc_ref[...]
    hn = h_ref[...] + _EPS * jnp.tanh(conv)
    hn_ref[...] = hn
    pr_ref[...] = jnp.dot(hn, wr_ref[...],
                          preferred_element_type=jnp.float32) + br_ref[...]


def _step(h, g1, g2, w0, w1, w2, b_conv, w_ro, b_ro, nrm):
    blk = 1280
    full = lambda i: (0, 0)
    rows = lambda i: (i, 0)
    return pl.pallas_call(
        _step_body,
        grid=(_NPAD // blk,),
        in_specs=[
            pl.BlockSpec((blk, _DH), rows),
            pl.BlockSpec((blk, _DH), rows),
            pl.BlockSpec((blk, _DH), rows),
            pl.BlockSpec((_DH, _DH), full),
            pl.BlockSpec((_DH, _DH), full),
            pl.BlockSpec((_DH, _DH), full),
            pl.BlockSpec((1, _DH), full),
            pl.BlockSpec((_DH, _DH), full),
            pl.BlockSpec((1, _DH), full),
            pl.BlockSpec((1, 1), full),
        ],
        out_specs=[
            pl.BlockSpec((blk, _DH), rows),
            pl.BlockSpec((blk, _DH), rows),
        ],
        out_shape=[
            jax.ShapeDtypeStruct((_NPAD, _DH), jnp.float32),
            jax.ShapeDtypeStruct((_NPAD, _DH), jnp.float32),
        ],
    )(h, g1, g2, w0, w1, w2, b_conv, w_ro, b_ro, nrm)


@functools.cache
def _make_gather_sum():
    return functools.partial(
        pl.kernel,
        out_type=jax.ShapeDtypeStruct((_NPAD, _DH), jnp.float32),
        mesh=plsc.VectorSubcoreMesh(core_axis_name="c", subcore_axis_name="s"),
        scratch_types=[
            pltpu.VMEM((_BPW * _KNN,), jnp.int32),
            pltpu.VMEM((2, _EPC, _DH), jnp.float32),
            pltpu.VMEM((_CHUNK, _DH), jnp.float32),
            pltpu.SemaphoreType.DMA,
            pltpu.SemaphoreType.DMA,
        ],
    )(_gather_sum_body)


def _gather_sum_body(table_hbm, idx_hbm, out_hbm, idx_v, rows_v, out_v,
                     sem0, sem1):
    wid = lax.axis_index("s") * 2 + lax.axis_index("c")
    base = wid * _BPW
    pltpu.sync_copy(idx_hbm.at[pl.ds(base * _KNN, _BPW * _KNN)], idx_v)
    sems = (sem0, sem1)

    def copies(c, buf):
        return [
            pltpu.make_async_copy(
                table_hbm.at[idx_v.at[pl.ds(c * _EPC + j * _GB, _GB)]],
                rows_v.at[buf, pl.ds(j * _GB, _GB)],
                sems[buf],
            )
            for j in range(_NGATH)
        ]

    def compute(c, buf):
        def node_body(n, carry):
            e0 = n * _KNN
            for j in range(_DH // 16):
                sl = pl.ds(j * 16, 16)
                acc = rows_v[buf, e0, sl]
                for e in range(1, _KNN):
                    acc = acc + rows_v[buf, e0 + e, sl]
                out_v[n, sl] = acc
            return carry

        lax.fori_loop(0, _CHUNK, node_body, 0)
        pltpu.sync_copy(out_v, out_hbm.at[pl.ds(base + c * _CHUNK, _CHUNK)])

    for cp in copies(0, 0):
        cp.start()

    def pair_body(p, carry):
        c0 = 2 * p
        for cp in copies(c0 + 1, 1):
            cp.start()
        for cp in copies(c0, 0):
            cp.wait()
        compute(c0, 0)

        @pl.when(p < _NCH // 2 - 1)
        def _():
            for cp in copies(c0 + 2, 0):
                cp.start()

        for cp in copies(c0 + 1, 1):
            cp.wait()
        compute(c0 + 1, 1)
        return carry

    lax.fori_loop(0, _NCH // 2, pair_body, 0)


def kernel(x, W_emb, b_emb, W_lins, b_conv, W_ro, b_ro):
    f32 = jnp.float32
    x_rows = jnp.zeros((_NPAD, 8), f32).at[:_N, :3].set(x)
    x_t = x_rows.T
    sq = jnp.sum(x * x, axis=1)
    sq_pad = jnp.zeros((_NPAD,), f32).at[:_N].set(sq)

    nbr = _knn(x_rows, x_t, sq_pad.reshape(_NPAD, 1), sq_pad.reshape(1, _NPAD))
    idx_flat = nbr[:, :_KNN].reshape(-1)

    dis = 1.0 / jnp.sqrt(f32(10.0))
    nrm = (dis * dis).reshape(1, 1)
    w_emb8 = jnp.zeros((8, _DH), f32).at[:3].set(W_emb)
    w0 = W_lins[0]
    w1 = W_lins[1]
    w2 = W_lins[2]
    w_ro_p = jnp.zeros((_DH, _DH), f32).at[:, :3].set(W_ro)
    b_ro_p = jnp.zeros((1, _DH), f32).at[0, :3].set(b_ro)
    b_conv_r = b_conv.reshape(1, _DH)
    b_emb_r = b_emb.reshape(1, _DH)

    h, x0 = _init(x_rows, w_emb8, b_emb_r, w_ro_p, b_ro_p)

    preds = []
    gather_sum = _make_gather_sum()
    for _t in range(_STEPS):
        g1 = gather_sum(h, idx_flat)
        g2 = gather_sum(g1, idx_flat)
        h, pr = _step(h, g1, g2, w0, w1, w2, b_conv_r, w_ro_p, b_ro_p, nrm)
        preds.append(pr[:_N, :3])

    y = preds[-1]
    return (y, h[:_N], x0[:_N, :3], jnp.stack(preds))


# speedup vs baseline: 11.9266x; 1.3702x over previous
import functools

import jax
import jax.numpy as jnp
from jax import lax
from jax.experimental import pallas as pl
from jax.experimental.pallas import tpu as pltpu
from jax.experimental.pallas import tpu_sc as plsc

_N = 10000
_NPAD = 10240
_DH = 128
_KNN = 10
_STEPS = 6
_EPS = 0.1

_ROWB = 128
_NBLK = _NPAD // _ROWB

_NW = 32
_BPW = _NPAD // _NW
_CHUNK = 32
_NCH = _BPW // _CHUNK
_EPC = _CHUNK * _KNN
_GB = 80
_NGATH = _EPC // _GB

_NTILE = _NPAD // 128
_TGRP = 8


def _knn_body(xr_ref, xt_ref, sqr_ref, sqc_ref, out_ref, d2_ref):
    i = pl.program_id(0)
    xr = xr_ref[...]
    xt = xt_ref[...]
    dot = jnp.dot(xr, xt, preferred_element_type=jnp.float32)
    d2 = (sqr_ref[...] + sqc_ref[...]) - 2.0 * dot
    col = lax.broadcasted_iota(jnp.int32, (_ROWB, _NPAD), 1)
    row = lax.broadcasted_iota(jnp.int32, (_ROWB, _NPAD), 0) + i * _ROWB
    inf = jnp.float32(jnp.inf)
    d2_ref[...] = jnp.where((col == row) | (col >= _N), inf, d2)

    lanef = lax.broadcasted_iota(jnp.int32, (_ROWB, 128), 1).astype(jnp.float32)
    out_col = lax.broadcasted_iota(jnp.int32, (_ROWB, 16), 1)
    big = jnp.float32(3e7)

    def tour_body(g, carry):
        m1, m2, m3, t1, t2, t3 = carry
        for tt in range(_TGRP):
            t = g * _TGRP + tt
            v = d2_ref[:, pl.ds(t * 128, 128)]
            tf = t.astype(jnp.float32)
            c1 = v < m1
            c2 = v < m2
            c3 = v < m3
            m3 = jnp.where(c2, m2, jnp.where(c3, v, m3))
            t3 = jnp.where(c2, t2, jnp.where(c3, tf, t3))
            m2 = jnp.where(c1, m1, jnp.where(c2, v, m2))
            t2 = jnp.where(c1, t1, jnp.where(c2, tf, t2))
            m1 = jnp.where(c1, v, m1)
            t1 = jnp.where(c1, tf, t1)
        return (m1, m2, m3, t1, t2, t3)

    finit = jnp.full((_ROWB, 128), inf)
    zinit = jnp.zeros((_ROWB, 128), jnp.float32)
    m1, m2, m3, t1, t2, t3 = lax.fori_loop(
        0, _NTILE // _TGRP, tour_body,
        (finit, finit, finit, zinit, zinit, zinit))
    g1 = t1 * 128.0 + lanef
    g2 = t2 * 128.0 + lanef
    g3 = t3 * 128.0 + lanef

    out = jnp.zeros((_ROWB, 16), jnp.int32)
    v10 = None
    for t in range(_KNN):
        mm = jnp.minimum(jnp.minimum(m1, m2), m3)
        m = jnp.min(mm, axis=1, keepdims=True)
        gsel = jnp.minimum(
            jnp.minimum(jnp.where(m1 == m, g1, big), jnp.where(m2 == m, g2, big)),
            jnp.where(m3 == m, g3, big))
        g = jnp.min(gsel, axis=1, keepdims=True)
        out = jnp.where(out_col == t, g.astype(jnp.int32), out)
        v10 = m
        m1 = jnp.where(g1 == g, inf, m1)
        m2 = jnp.where(g2 == g, inf, m2)
        m3 = jnp.where(g3 == g, inf, m3)
    out_ref[...] = out

    def cnt_body(g, cnt):
        for tt in range(_TGRP):
            t = g * _TGRP + tt
            v = d2_ref[:, pl.ds(t * 128, 128)]
            cnt = cnt + jnp.where(v <= v10, 1.0, 0.0)
        return cnt
    cnt = lax.fori_loop(0, _NTILE // _TGRP, cnt_body, zinit)
    bad = jnp.max(cnt) > 3.0

    @pl.when(bad)
    def _fallback():
        colf = col.astype(jnp.float32)

        def ext_body(t, out_fb):
            d2c = d2_ref[...]
            m = jnp.min(d2c, axis=1, keepdims=True)
            idx = jnp.min(jnp.where(d2c == m, colf, big), axis=1, keepdims=True)
            out_fb = jnp.where(out_col == t, idx.astype(jnp.int32), out_fb)
            d2_ref[...] = jnp.where(colf == idx, inf, d2c)
            return out_fb

        out_fb = lax.fori_loop(0, _KNN, ext_body, jnp.zeros((_ROWB, 16), jnp.int32))
        out_ref[...] = out_fb


def _knn(x_rows, x_t, sq_col, sq_row):
    return pl.pallas_call(
        _knn_body,
        grid=(_NBLK,),
        in_specs=[
            pl.BlockSpec((_ROWB, 8), lambda i: (i, 0)),
            pl.BlockSpec((8, _NPAD), lambda i: (0, 0)),
            pl.BlockSpec((_ROWB, 1), lambda i: (i, 0)),
            pl.BlockSpec((1, _NPAD), lambda i: (0, 0)),
        ],
        out_specs=pl.BlockSpec((_ROWB, 16), lambda i: (i, 0)),
        out_shape=jax.ShapeDtypeStruct((_NPAD, 16), jnp.int32),
        scratch_shapes=[pltpu.VMEM((_ROWB, _NPAD), jnp.float32)],
    )(x_rows, x_t, sq_col, sq_row)


def _init_body(xr_ref, we_ref, be_ref, wr_ref, br_ref, h_ref, x0_ref):
    h = jnp.dot(xr_ref[...], we_ref[...],
                preferred_element_type=jnp.float32) + be_ref[...]
    h_ref[...] = h
    x0_ref[...] = jnp.dot(h, wr_ref[...],
                          preferred_element_type=jnp.float32) + br_ref[...]


def _init(x_rows, w_emb8, b_emb, w_ro, b_ro):
    blk = 1280
    full = lambda i: (0, 0)
    return pl.pallas_call(
        _init_body,
        grid=(_NPAD // blk,),
        in_specs=[
            pl.BlockSpec((blk, 8), lambda i: (i, 0)),
            pl.BlockSpec((8, _DH), full),
            pl.BlockSpec((1, _DH), full),
            pl.BlockSpec((_DH, _DH), full),
            pl.BlockSpec((1, _DH), full),
        ],
        out_specs=[
            pl.BlockSpec((blk, _DH), lambda i: (i, 0)),
            pl.BlockSpec((blk, _DH), lambda i: (i, 0)),
        ],
        out_shape=[
            jax.ShapeDtypeStruct((_NPAD, _DH), jnp.float32),
            jax.ShapeDtypeStruct((_NPAD, _DH), jnp.float32),
        ],
    )(x_rows, w_emb8, b_emb, w_ro, b_ro)


def _step_body(h_ref, g1_ref, g2_ref, w0_ref, w1_ref, w2_ref, bc_ref,
               wr_ref, br_ref, nrm_ref, hn_ref, pr_ref):
    nrm = nrm_ref[0, 0]
    x1 = g1_ref[...] * nrm
    x2 = (g2_ref[...] * nrm) * nrm
    conv = jnp.dot(h_ref[...], w0_ref[...],
                   preferred_element_type=jnp.float32)
    conv = conv + jnp.dot(x1, w1_ref[...],
                          preferred_element_type=jnp.float32)
    conv = conv + jnp.dot(x2, w2_ref[...],
                          preferred_element_type=jnp.float32)
    conv = conv + bc_ref[...]
    hn = h_ref[...] + _EPS * jnp.tanh(conv)
    hn_ref[...] = hn
    pr_ref[...] = jnp.dot(hn, wr_ref[...],
                          preferred_element_type=jnp.float32) + br_ref[...]


def _step(h, g1, g2, w0, w1, w2, b_conv, w_ro, b_ro, nrm):
    blk = 1280
    full = lambda i: (0, 0)
    rows = lambda i: (i, 0)
    return pl.pallas_call(
        _step_body,
        grid=(_NPAD // blk,),
        in_specs=[
            pl.BlockSpec((blk, _DH), rows),
            pl.BlockSpec((blk, _DH), rows),
            pl.BlockSpec((blk, _DH), rows),
            pl.BlockSpec((_DH, _DH), full),
            pl.BlockSpec((_DH, _DH), full),
            pl.BlockSpec((_DH, _DH), full),
            pl.BlockSpec((1, _DH), full),
            pl.BlockSpec((_DH, _DH), full),
            pl.BlockSpec((1, _DH), full),
            pl.BlockSpec((1, 1), full),
        ],
        out_specs=[
            pl.BlockSpec((blk, _DH), rows),
            pl.BlockSpec((blk, _DH), rows),
        ],
        out_shape=[
            jax.ShapeDtypeStruct((_NPAD, _DH), jnp.float32),
            jax.ShapeDtypeStruct((_NPAD, _DH), jnp.float32),
        ],
    )(h, g1, g2, w0, w1, w2, b_conv, w_ro, b_ro, nrm)


@functools.cache
def _make_gather_sum():
    return functools.partial(
        pl.kernel,
        out_type=jax.ShapeDtypeStruct((_NPAD, _DH), jnp.float32),
        mesh=plsc.VectorSubcoreMesh(core_axis_name="c", subcore_axis_name="s"),
        scratch_types=[
            pltpu.VMEM((_BPW * _KNN,), jnp.int32),
            pltpu.VMEM((2, _EPC, _DH), jnp.float32),
            pltpu.VMEM((_CHUNK, _DH), jnp.float32),
            pltpu.SemaphoreType.DMA,
            pltpu.SemaphoreType.DMA,
        ],
    )(_gather_sum_body)


def _gather_sum_body(table_hbm, idx_hbm, out_hbm, idx_v, rows_v, out_v,
                     sem0, sem1):
    wid = lax.axis_index("s") * 2 + lax.axis_index("c")
    base = wid * _BPW
    pltpu.sync_copy(idx_hbm.at[pl.ds(base * _KNN, _BPW * _KNN)], idx_v)
    sems = (sem0, sem1)

    def copies(c, buf):
        return [
            pltpu.make_async_copy(
                table_hbm.at[idx_v.at[pl.ds(c * _EPC + j * _GB, _GB)]],
                rows_v.at[buf, pl.ds(j * _GB, _GB)],
                sems[buf],
            )
            for j in range(_NGATH)
        ]

    def compute(c, buf):
        def node_body(n, carry):
            e0 = n * _KNN
            for j in range(_DH // 16):
                sl = pl.ds(j * 16, 16)
                acc = rows_v[buf, e0, sl]
                for e in range(1, _KNN):
                    acc = acc + rows_v[buf, e0 + e, sl]
                out_v[n, sl] = acc
            return carry

        lax.fori_loop(0, _CHUNK, node_body, 0)
        pltpu.sync_copy(out_v, out_hbm.at[pl.ds(base + c * _CHUNK, _CHUNK)])

    for cp in copies(0, 0):
        cp.start()

    def pair_body(p, carry):
        c0 = 2 * p
        for cp in copies(c0 + 1, 1):
            cp.start()
        for cp in copies(c0, 0):
            cp.wait()
        compute(c0, 0)

        @pl.when(p < _NCH // 2 - 1)
        def _():
            for cp in copies(c0 + 2, 0):
                cp.start()

        for cp in copies(c0 + 1, 1):
            cp.wait()
        compute(c0 + 1, 1)
        return carry

    lax.fori_loop(0, _NCH // 2, pair_body, 0)


def kernel(x, W_emb, b_emb, W_lins, b_conv, W_ro, b_ro):
    f32 = jnp.float32
    x_rows = jnp.zeros((_NPAD, 8), f32).at[:_N, :3].set(x)
    x_t = x_rows.T
    sq = jnp.sum(x * x, axis=1)
    sq_pad = jnp.zeros((_NPAD,), f32).at[:_N].set(sq)

    nbr = _knn(x_rows, x_t, sq_pad.reshape(_NPAD, 1), sq_pad.reshape(1, _NPAD))
    idx_flat = nbr[:, :_KNN].reshape(-1)

    dis = 1.0 / jnp.sqrt(f32(10.0))
    nrm = (dis * dis).reshape(1, 1)
    w_emb8 = jnp.zeros((8, _DH), f32).at[:3].set(W_emb)
    w0 = W_lins[0]
    w1 = W_lins[1]
    w2 = W_lins[2]
    w_ro_p = jnp.zeros((_DH, _DH), f32).at[:, :3].set(W_ro)
    b_ro_p = jnp.zeros((1, _DH), f32).at[0, :3].set(b_ro)
    b_conv_r = b_conv.reshape(1, _DH)
    b_emb_r = b_emb.reshape(1, _DH)

    h, x0 = _init(x_rows, w_emb8, b_emb_r, w_ro_p, b_ro_p)

    preds = []
    gather_sum = _make_gather_sum()
    for _t in range(_STEPS):
        g1 = gather_sum(h, idx_flat)
        g2 = gather_sum(g1, idx_flat)
        h, pr = _step(h, g1, g2, w0, w1, w2, b_conv_r, w_ro_p, b_ro_p, nrm)
        preds.append(pr[:_N, :3])

    y = preds[-1]
    return (y, h[:_N], x0[:_N, :3], jnp.stack(preds))


# speedup vs baseline: 13.2639x; 1.1121x over previous
import functools

import jax
import jax.numpy as jnp
from jax import lax
from jax.experimental import pallas as pl
from jax.experimental.pallas import tpu as pltpu
from jax.experimental.pallas import tpu_sc as plsc

_N = 10000
_NPAD = 10240
_DH = 128
_KNN = 10
_STEPS = 6
_EPS = 0.1

_ROWB = 128
_NBLK = _NPAD // _ROWB

_NW = 32
_BPW = _NPAD // _NW
_CHUNK = 32
_NCH = _BPW // _CHUNK
_EPC = _CHUNK * _KNN
_GB = 80
_NGATH = _EPC // _GB

_NTILE = _NPAD // 128
_TGRP = 8


def _knn_body(xr_ref, xt_ref, sqr_ref, sqc_ref, out_ref, d2_ref):
    i = pl.program_id(0)
    xr = xr_ref[...]
    xt = xt_ref[...]
    dot = jnp.dot(xr, xt, preferred_element_type=jnp.float32)
    d2 = (sqr_ref[...] + sqc_ref[...]) - 2.0 * dot
    col = lax.broadcasted_iota(jnp.int32, (_ROWB, _NPAD), 1)
    row = lax.broadcasted_iota(jnp.int32, (_ROWB, _NPAD), 0) + i * _ROWB
    inf = jnp.float32(jnp.inf)
    d2_ref[...] = jnp.where((col == row) | (col >= _N), inf, d2)

    lanef = lax.broadcasted_iota(jnp.int32, (_ROWB, 128), 1).astype(jnp.float32)
    out_col = lax.broadcasted_iota(jnp.int32, (_ROWB, 16), 1)
    big = jnp.float32(3e7)

    finit = jnp.full((_ROWB, 128), inf)
    zinit = jnp.zeros((_ROWB, 128), jnp.float32)
    m1 = m2 = m3 = m4 = finit
    t1 = t2 = t3 = zinit
    for t in range(_NTILE):
        v = d2_ref[:, pl.ds(t * 128, 128)]
        tf = jnp.float32(t)
        c1 = v < m1
        c2 = v < m2
        c3 = v < m3
        c4 = v < m4
        m4 = jnp.where(c3, m3, jnp.where(c4, v, m4))
        m3 = jnp.where(c2, m2, jnp.where(c3, v, m3))
        t3 = jnp.where(c2, t2, jnp.where(c3, tf, t3))
        m2 = jnp.where(c1, m1, jnp.where(c2, v, m2))
        t2 = jnp.where(c1, t1, jnp.where(c2, tf, t2))
        m1 = jnp.where(c1, v, m1)
        t1 = jnp.where(c1, tf, t1)
    g1 = t1 * 128.0 + lanef
    g2 = t2 * 128.0 + lanef
    g3 = t3 * 128.0 + lanef

    out = jnp.zeros((_ROWB, 16), jnp.int32)
    v10 = None
    for t in range(_KNN):
        mm = jnp.minimum(jnp.minimum(m1, m2), m3)
        m = jnp.min(mm, axis=1, keepdims=True)
        gsel = jnp.minimum(
            jnp.minimum(jnp.where(m1 == m, g1, big), jnp.where(m2 == m, g2, big)),
            jnp.where(m3 == m, g3, big))
        g = jnp.min(gsel, axis=1, keepdims=True)
        out = jnp.where(out_col == t, g.astype(jnp.int32), out)
        v10 = m
        m1 = jnp.where(g1 == g, inf, m1)
        m2 = jnp.where(g2 == g, inf, m2)
        m3 = jnp.where(g3 == g, inf, m3)
    out_ref[...] = out

    bad = jnp.min(jnp.where(m4 <= v10, 0.0, 1.0)) < 0.5

    @pl.when(bad)
    def _fallback():
        colf = col.astype(jnp.float32)

        def ext_body(t, out_fb):
            d2c = d2_ref[...]
            m = jnp.min(d2c, axis=1, keepdims=True)
            idx = jnp.min(jnp.where(d2c == m, colf, big), axis=1, keepdims=True)
            out_fb = jnp.where(out_col == t, idx.astype(jnp.int32), out_fb)
            d2_ref[...] = jnp.where(colf == idx, inf, d2c)
            return out_fb

        out_fb = lax.fori_loop(0, _KNN, ext_body, jnp.zeros((_ROWB, 16), jnp.int32))
        out_ref[...] = out_fb


def _knn(x_rows, x_t, sq_col, sq_row):
    return pl.pallas_call(
        _knn_body,
        grid=(_NBLK,),
        in_specs=[
            pl.BlockSpec((_ROWB, 8), lambda i: (i, 0)),
            pl.BlockSpec((8, _NPAD), lambda i: (0, 0)),
            pl.BlockSpec((_ROWB, 1), lambda i: (i, 0)),
            pl.BlockSpec((1, _NPAD), lambda i: (0, 0)),
        ],
        out_specs=pl.BlockSpec((_ROWB, 16), lambda i: (i, 0)),
        out_shape=jax.ShapeDtypeStruct((_NPAD, 16), jnp.int32),
        scratch_shapes=[pltpu.VMEM((_ROWB, _NPAD), jnp.float32)],
    )(x_rows, x_t, sq_col, sq_row)


def _init_body(xr_ref, we_ref, be_ref, wr_ref, br_ref, h_ref, x0_ref):
    h = jnp.dot(xr_ref[...], we_ref[...],
                preferred_element_type=jnp.float32) + be_ref[...]
    h_ref[...] = h
    x0_ref[...] = jnp.dot(h, wr_ref[...],
                          preferred_element_type=jnp.float32) + br_ref[...]


def _init(x_rows, w_emb8, b_emb, w_ro, b_ro):
    blk = 1280
    full = lambda i: (0, 0)
    return pl.pallas_call(
        _init_body,
        grid=(_NPAD // blk,),
        in_specs=[
            pl.BlockSpec((blk, 8), lambda i: (i, 0)),
            pl.BlockSpec((8, _DH), full),
            pl.BlockSpec((1, _DH), full),
            pl.BlockSpec((_DH, _DH), full),
            pl.BlockSpec((1, _DH), full),
        ],
        out_specs=[
            pl.BlockSpec((blk, _DH), lambda i: (i, 0)),
            pl.BlockSpec((blk, _DH), lambda i: (i, 0)),
        ],
        out_shape=[
            jax.ShapeDtypeStruct((_NPAD, _DH), jnp.float32),
            jax.ShapeDtypeStruct((_NPAD, _DH), jnp.float32),
        ],
    )(x_rows, w_emb8, b_emb, w_ro, b_ro)


def _step_body(h_ref, g1_ref, g2_ref, w0_ref, w1_ref, w2_ref, bc_ref,
               wr_ref, br_ref, nrm_ref, hn_ref, pr_ref):
    nrm = nrm_ref[0, 0]
    x1 = g1_ref[...] * nrm
    x2 = (g2_ref[...] * nrm) * nrm
    conv = jnp.dot(h_ref[...], w0_ref[...],
                   preferred_element_type=jnp.float32)
    conv = conv + jnp.dot(x1, w1_ref[...],
                          preferred_element_type=jnp.float32)
    conv = conv + jnp.dot(x2, w2_ref[...],
                          preferred_element_type=jnp.float32)
    conv = conv + bc_ref[...]
    hn = h_ref[...] + _EPS * jnp.tanh(conv)
    hn_ref[...] = hn
    pr_ref[...] = jnp.dot(hn, wr_ref[...],
                          preferred_element_type=jnp.float32) + br_ref[...]


def _step(h, g1, g2, w0, w1, w2, b_conv, w_ro, b_ro, nrm):
    blk = 1280
    full = lambda i: (0, 0)
    rows = lambda i: (i, 0)
    return pl.pallas_call(
        _step_body,
        grid=(_NPAD // blk,),
        in_specs=[
            pl.BlockSpec((blk, _DH), rows),
            pl.BlockSpec((blk, _DH), rows),
            pl.BlockSpec((blk, _DH), rows),
            pl.BlockSpec((_DH, _DH), full),
            pl.BlockSpec((_DH, _DH), full),
            pl.BlockSpec((_DH, _DH), full),
            pl.BlockSpec((1, _DH), full),
            pl.BlockSpec((_DH, _DH), full),
            pl.BlockSpec((1, _DH), full),
            pl.BlockSpec((1, 1), full),
        ],
        out_specs=[
            pl.BlockSpec((blk, _DH), rows),
            pl.BlockSpec((blk, _DH), rows),
        ],
        out_shape=[
            jax.ShapeDtypeStruct((_NPAD, _DH), jnp.float32),
            jax.ShapeDtypeStruct((_NPAD, _DH), jnp.float32),
        ],
    )(h, g1, g2, w0, w1, w2, b_conv, w_ro, b_ro, nrm)


@functools.cache
def _make_gather_sum():
    return functools.partial(
        pl.kernel,
        out_type=jax.ShapeDtypeStruct((_NPAD, _DH), jnp.float32),
        mesh=plsc.VectorSubcoreMesh(core_axis_name="c", subcore_axis_name="s"),
        scratch_types=[
            pltpu.VMEM((_BPW * _KNN,), jnp.int32),
            pltpu.VMEM((2, _EPC, _DH), jnp.float32),
            pltpu.VMEM((_CHUNK, _DH), jnp.float32),
            pltpu.SemaphoreType.DMA,
            pltpu.SemaphoreType.DMA,
        ],
    )(_gather_sum_body)


def _gather_sum_body(table_hbm, idx_hbm, out_hbm, idx_v, rows_v, out_v,
                     sem0, sem1):
    wid = lax.axis_index("s") * 2 + lax.axis_index("c")
    base = wid * _BPW
    pltpu.sync_copy(idx_hbm.at[pl.ds(base * _KNN, _BPW * _KNN)], idx_v)
    sems = (sem0, sem1)

    def copies(c, buf):
        return [
            pltpu.make_async_copy(
                table_hbm.at[idx_v.at[pl.ds(c * _EPC + j * _GB, _GB)]],
                rows_v.at[buf, pl.ds(j * _GB, _GB)],
                sems[buf],
            )
            for j in range(_NGATH)
        ]

    def compute(c, buf):
        def node_body(n, carry):
            e0 = n * _KNN
            for j in range(_DH // 16):
                sl = pl.ds(j * 16, 16)
                acc = rows_v[buf, e0, sl]
                for e in range(1, _KNN):
                    acc = acc + rows_v[buf, e0 + e, sl]
                out_v[n, sl] = acc
            return carry

        lax.fori_loop(0, _CHUNK, node_body, 0)
        pltpu.sync_copy(out_v, out_hbm.at[pl.ds(base + c * _CHUNK, _CHUNK)])

    for cp in copies(0, 0):
        cp.start()

    def pair_body(p, carry):
        c0 = 2 * p
        for cp in copies(c0 + 1, 1):
            cp.start()
        for cp in copies(c0, 0):
            cp.wait()
        compute(c0, 0)

        @pl.when(p < _NCH // 2 - 1)
        def _():
            for cp in copies(c0 + 2, 0):
                cp.start()

        for cp in copies(c0 + 1, 1):
            cp.wait()
        compute(c0 + 1, 1)
        return carry

    lax.fori_loop(0, _NCH // 2, pair_body, 0)


def kernel(x, W_emb, b_emb, W_lins, b_conv, W_ro, b_ro):
    f32 = jnp.float32
    x_rows = jnp.zeros((_NPAD, 8), f32).at[:_N, :3].set(x)
    x_t = x_rows.T
    sq = jnp.sum(x * x, axis=1)
    sq_pad = jnp.zeros((_NPAD,), f32).at[:_N].set(sq)

    nbr = _knn(x_rows, x_t, sq_pad.reshape(_NPAD, 1), sq_pad.reshape(1, _NPAD))
    idx_flat = nbr[:, :_KNN].reshape(-1)

    dis = 1.0 / jnp.sqrt(f32(10.0))
    nrm = (dis * dis).reshape(1, 1)
    w_emb8 = jnp.zeros((8, _DH), f32).at[:3].set(W_emb)
    w0 = W_lins[0]
    w1 = W_lins[1]
    w2 = W_lins[2]
    w_ro_p = jnp.zeros((_DH, _DH), f32).at[:, :3].set(W_ro)
    b_ro_p = jnp.zeros((1, _DH), f32).at[0, :3].set(b_ro)
    b_conv_r = b_conv.reshape(1, _DH)
    b_emb_r = b_emb.reshape(1, _DH)

    h, x0 = _init(x_rows, w_emb8, b_emb_r, w_ro_p, b_ro_p)

    preds = []
    gather_sum = _make_gather_sum()
    for _t in range(_STEPS):
        g1 = gather_sum(h, idx_flat)
        g2 = gather_sum(g1, idx_flat)
        h, pr = _step(h, g1, g2, w0, w1, w2, b_conv_r, w_ro_p, b_ro_p, nrm)
        preds.append(pr[:_N, :3])

    y = preds[-1]
    return (y, h[:_N], x0[:_N, :3], jnp.stack(preds))


# speedup vs baseline: 13.4533x; 1.0143x over previous
import functools

import jax
import jax.numpy as jnp
from jax import lax
from jax.experimental import pallas as pl
from jax.experimental.pallas import tpu as pltpu
from jax.experimental.pallas import tpu_sc as plsc

_N = 10000
_NPAD = 10240
_DH = 128
_KNN = 10
_STEPS = 6
_EPS = 0.1

_ROWB = 128
_NBLK = _NPAD // _ROWB

_NW = 32
_BPW = _NPAD // _NW
_CHUNK = 32
_NCH = _BPW // _CHUNK
_EPC = _CHUNK * _KNN
_GB = 80
_NGATH = _EPC // _GB

_NTILE = _NPAD // 128
_TGRP = 8


def _knn_body(xr_ref, xt_ref, sqr_ref, sqc_ref, out_ref, d2_ref):
    i = pl.program_id(0)
    xr = xr_ref[...]
    xt = xt_ref[...]
    dot = jnp.dot(xr, xt, preferred_element_type=jnp.float32)
    d2 = (sqr_ref[...] + sqc_ref[...]) - 2.0 * dot
    col = lax.broadcasted_iota(jnp.int32, (_ROWB, _NPAD), 1)
    row = lax.broadcasted_iota(jnp.int32, (_ROWB, _NPAD), 0) + i * _ROWB
    inf = jnp.float32(jnp.inf)
    d2_ref[...] = jnp.where((col == row) | (col >= _N), inf, d2)

    lanef = lax.broadcasted_iota(jnp.int32, (_ROWB, 128), 1).astype(jnp.float32)
    out_col = lax.broadcasted_iota(jnp.int32, (_ROWB, 16), 1)
    big = jnp.float32(3e7)

    finit = jnp.full((_ROWB, 128), inf)
    zinit = jnp.zeros((_ROWB, 128), jnp.float32)
    m1 = m2 = m3 = m4 = finit
    t1 = t2 = t3 = zinit
    for t in range(_NTILE):
        v = d2_ref[:, pl.ds(t * 128, 128)]
        tf = jnp.float32(t)
        c1 = v < m1
        c2 = v < m2
        c3 = v < m3
        c4 = v < m4
        m4 = jnp.where(c3, m3, jnp.where(c4, v, m4))
        m3 = jnp.where(c2, m2, jnp.where(c3, v, m3))
        t3 = jnp.where(c2, t2, jnp.where(c3, tf, t3))
        m2 = jnp.where(c1, m1, jnp.where(c2, v, m2))
        t2 = jnp.where(c1, t1, jnp.where(c2, tf, t2))
        m1 = jnp.where(c1, v, m1)
        t1 = jnp.where(c1, tf, t1)
    g1 = t1 * 128.0 + lanef
    g2 = t2 * 128.0 + lanef
    g3 = t3 * 128.0 + lanef

    out = jnp.zeros((_ROWB, 16), jnp.int32)
    v10 = None
    for t in range(_KNN):
        mm = jnp.minimum(jnp.minimum(m1, m2), m3)
        m = jnp.min(mm, axis=1, keepdims=True)
        gsel = jnp.minimum(
            jnp.minimum(jnp.where(m1 == m, g1, big), jnp.where(m2 == m, g2, big)),
            jnp.where(m3 == m, g3, big))
        g = jnp.min(gsel, axis=1, keepdims=True)
        out = jnp.where(out_col == t, g.astype(jnp.int32), out)
        v10 = m
        m1 = jnp.where(g1 == g, inf, m1)
        m2 = jnp.where(g2 == g, inf, m2)
        m3 = jnp.where(g3 == g, inf, m3)
    out_ref[...] = out

    bad = jnp.min(jnp.where(m4 <= v10, 0.0, 1.0)) < 0.5

    @pl.when(bad)
    def _fallback():
        colf = col.astype(jnp.float32)

        def ext_body(t, out_fb):
            d2c = d2_ref[...]
            m = jnp.min(d2c, axis=1, keepdims=True)
            idx = jnp.min(jnp.where(d2c == m, colf, big), axis=1, keepdims=True)
            out_fb = jnp.where(out_col == t, idx.astype(jnp.int32), out_fb)
            d2_ref[...] = jnp.where(colf == idx, inf, d2c)
            return out_fb

        out_fb = lax.fori_loop(0, _KNN, ext_body, jnp.zeros((_ROWB, 16), jnp.int32))
        out_ref[...] = out_fb


def _knn(x_rows, x_t, sq_col, sq_row):
    return pl.pallas_call(
        _knn_body,
        grid=(_NBLK,),
        in_specs=[
            pl.BlockSpec((_ROWB, 8), lambda i: (i, 0)),
            pl.BlockSpec((8, _NPAD), lambda i: (0, 0)),
            pl.BlockSpec((_ROWB, 1), lambda i: (i, 0)),
            pl.BlockSpec((1, _NPAD), lambda i: (0, 0)),
        ],
        out_specs=pl.BlockSpec((_ROWB, 16), lambda i: (i, 0)),
        out_shape=jax.ShapeDtypeStruct((_NPAD, 16), jnp.int32),
        scratch_shapes=[pltpu.VMEM((_ROWB, _NPAD), jnp.float32)],
    )(x_rows, x_t, sq_col, sq_row)


def _init_body(xr_ref, we_ref, be_ref, wr_ref, br_ref, h_ref, x0_ref):
    h = jnp.dot(xr_ref[...], we_ref[...],
                preferred_element_type=jnp.float32) + be_ref[...]
    h_ref[...] = h
    x0_ref[...] = jnp.dot(h, wr_ref[...],
                          preferred_element_type=jnp.float32) + br_ref[...]


def _init(x_rows, w_emb8, b_emb, w_ro, b_ro):
    blk = 1280
    full = lambda i: (0, 0)
    return pl.pallas_call(
        _init_body,
        grid=(_NPAD // blk,),
        in_specs=[
            pl.BlockSpec((blk, 8), lambda i: (i, 0)),
            pl.BlockSpec((8, _DH), full),
            pl.BlockSpec((1, _DH), full),
            pl.BlockSpec((_DH, _DH), full),
            pl.BlockSpec((1, _DH), full),
        ],
        out_specs=[
            pl.BlockSpec((blk, _DH), lambda i: (i, 0)),
            pl.BlockSpec((blk, _DH), lambda i: (i, 0)),
        ],
        out_shape=[
            jax.ShapeDtypeStruct((_NPAD, _DH), jnp.float32),
            jax.ShapeDtypeStruct((_NPAD, _DH), jnp.float32),
        ],
    )(x_rows, w_emb8, b_emb, w_ro, b_ro)


def _step_body(h_ref, g1_ref, g2_ref, w0_ref, w1_ref, w2_ref, bc_ref,
               wr_ref, br_ref, nrm_ref, hn_ref, pr_ref):
    nrm = nrm_ref[0, 0]
    x1 = g1_ref[...] * nrm
    x2 = (g2_ref[...] * nrm) * nrm
    conv = jnp.dot(h_ref[...], w0_ref[...],
                   preferred_element_type=jnp.float32)
    conv = conv + jnp.dot(x1, w1_ref[...],
                          preferred_element_type=jnp.float32)
    conv = conv + jnp.dot(x2, w2_ref[...],
                          preferred_element_type=jnp.float32)
    conv = conv + bc_ref[...]
    hn = h_ref[...] + _EPS * jnp.tanh(conv)
    hn_ref[...] = hn
    pr_ref[...] = jnp.dot(hn, wr_ref[...],
                          preferred_element_type=jnp.float32) + br_ref[...]


def _step(h, g1, g2, w0, w1, w2, b_conv, w_ro, b_ro, nrm):
    blk = 1280
    full = lambda i: (0, 0)
    rows = lambda i: (i, 0)
    return pl.pallas_call(
        _step_body,
        grid=(_NPAD // blk,),
        in_specs=[
            pl.BlockSpec((blk, _DH), rows),
            pl.BlockSpec((blk, _DH), rows),
            pl.BlockSpec((blk, _DH), rows),
            pl.BlockSpec((_DH, _DH), full),
            pl.BlockSpec((_DH, _DH), full),
            pl.BlockSpec((_DH, _DH), full),
            pl.BlockSpec((1, _DH), full),
            pl.BlockSpec((_DH, _DH), full),
            pl.BlockSpec((1, _DH), full),
            pl.BlockSpec((1, 1), full),
        ],
        out_specs=[
            pl.BlockSpec((blk, _DH), rows),
            pl.BlockSpec((blk, _DH), rows),
        ],
        out_shape=[
            jax.ShapeDtypeStruct((_NPAD, _DH), jnp.float32),
            jax.ShapeDtypeStruct((_NPAD, _DH), jnp.float32),
        ],
    )(h, g1, g2, w0, w1, w2, b_conv, w_ro, b_ro, nrm)


@functools.cache
def _make_gather_sum():
    return functools.partial(
        pl.kernel,
        out_type=jax.ShapeDtypeStruct((_NPAD, _DH), jnp.float32),
        mesh=plsc.VectorSubcoreMesh(core_axis_name="c", subcore_axis_name="s"),
        scratch_types=[
            pltpu.VMEM((_BPW * _KNN,), jnp.int32),
            pltpu.VMEM((2, _EPC, _DH), jnp.float32),
            pltpu.VMEM((2, _CHUNK, _DH), jnp.float32),
            pltpu.SemaphoreType.DMA,
            pltpu.SemaphoreType.DMA,
            pltpu.SemaphoreType.DMA,
            pltpu.SemaphoreType.DMA,
        ],
    )(_gather_sum_body)


def _gather_sum_body(table_hbm, idx_hbm, out_hbm, idx_v, rows_v, out_v,
                     sem0, sem1, semo0, semo1):
    wid = lax.axis_index("s") * 2 + lax.axis_index("c")
    base = wid * _BPW
    pltpu.sync_copy(idx_hbm.at[pl.ds(base * _KNN, _BPW * _KNN)], idx_v)
    sems = (sem0, sem1)
    semos = (semo0, semo1)

    def copies(c, buf):
        return [
            pltpu.make_async_copy(
                table_hbm.at[idx_v.at[pl.ds(c * _EPC + j * _GB, _GB)]],
                rows_v.at[buf, pl.ds(j * _GB, _GB)],
                sems[buf],
            )
            for j in range(_NGATH)
        ]

    def out_copy(c, buf):
        return pltpu.make_async_copy(
            out_v.at[buf],
            out_hbm.at[pl.ds(base + c * _CHUNK, _CHUNK)],
            semos[buf],
        )

    def compute(c, buf, p):
        @pl.when(p > 0)
        def _():
            out_copy(c - 2, buf).wait()

        def node_body(n, carry):
            e0 = n * _KNN
            for j in range(_DH // 16):
                sl = pl.ds(j * 16, 16)
                acc = rows_v[buf, e0, sl]
                for e in range(1, _KNN):
                    acc = acc + rows_v[buf, e0 + e, sl]
                out_v[buf, n, sl] = acc
            return carry

        lax.fori_loop(0, _CHUNK, node_body, 0)
        out_copy(c, buf).start()

    for cp in copies(0, 0):
        cp.start()

    def pair_body(p, carry):
        c0 = 2 * p
        for cp in copies(c0 + 1, 1):
            cp.start()
        for cp in copies(c0, 0):
            cp.wait()
        compute(c0, 0, p)

        @pl.when(p < _NCH // 2 - 1)
        def _():
            for cp in copies(c0 + 2, 0):
                cp.start()

        for cp in copies(c0 + 1, 1):
            cp.wait()
        compute(c0 + 1, 1, p)
        return carry

    lax.fori_loop(0, _NCH // 2, pair_body, 0)
    out_copy(_NCH - 2, 0).wait()
    out_copy(_NCH - 1, 1).wait()


def kernel(x, W_emb, b_emb, W_lins, b_conv, W_ro, b_ro):
    f32 = jnp.float32
    x_rows = jnp.zeros((_NPAD, 8), f32).at[:_N, :3].set(x)
    x_t = x_rows.T
    sq = jnp.sum(x * x, axis=1)
    sq_pad = jnp.zeros((_NPAD,), f32).at[:_N].set(sq)

    nbr = _knn(x_rows, x_t, sq_pad.reshape(_NPAD, 1), sq_pad.reshape(1, _NPAD))
    idx_flat = nbr[:, :_KNN].reshape(-1)

    dis = 1.0 / jnp.sqrt(f32(10.0))
    nrm = (dis * dis).reshape(1, 1)
    w_emb8 = jnp.zeros((8, _DH), f32).at[:3].set(W_emb)
    w0 = W_lins[0]
    w1 = W_lins[1]
    w2 = W_lins[2]
    w_ro_p = jnp.zeros((_DH, _DH), f32).at[:, :3].set(W_ro)
    b_ro_p = jnp.zeros((1, _DH), f32).at[0, :3].set(b_ro)
    b_conv_r = b_conv.reshape(1, _DH)
    b_emb_r = b_emb.reshape(1, _DH)

    h, x0 = _init(x_rows, w_emb8, b_emb_r, w_ro_p, b_ro_p)

    preds = []
    gather_sum = _make_gather_sum()
    for _t in range(_STEPS):
        g1 = gather_sum(h, idx_flat)
        g2 = gather_sum(g1, idx_flat)
        h, pr = _step(h, g1, g2, w0, w1, w2, b_conv_r, w_ro_p, b_ro_p, nrm)
        preds.append(pr[:_N, :3])

    y = preds[-1]
    return (y, h[:_N], x0[:_N, :3], jnp.stack(preds))
